# baseline (speedup 1.0000x reference)
import functools

import jax
import jax.numpy as jnp
from jax.experimental import pallas as pl

B, S, D = 1, 2048, 2048
H = D // 2
E = 8
K = 2
R = 8
DO = 2048
EPS = 1e-5
ER = E * R
TT = 1024

_HI = jax.lax.Precision.HIGHEST
_DEF = jax.lax.Precision.DEFAULT


def _dot(a, b, prec=_DEF):
    return jnp.dot(a, b, precision=prec, preferred_element_type=jnp.float32)


def _fused_kernel(x_ref, w1_ref, b1_ref, w2_ref, b2_ref, a2d_ref, gain_ref,
                  bias_ref, b2d_ref, out_ref):
    xt = x_ref[...]
    h = jax.nn.gelu(_dot(xt, w1_ref[...]) + b1_ref[...])
    logits = _dot(h, w2_ref[...]) + b2_ref[...]
    idx = jax.lax.broadcasted_iota(jnp.int32, (TT, E), 1)
    v1 = jnp.max(logits, axis=1, keepdims=True)
    i1 = jnp.min(jnp.where(logits >= v1, idx, E), axis=1, keepdims=True)
    sel1 = idx == i1
    ml = jnp.where(sel1, -jnp.inf, logits)
    v2 = jnp.max(ml, axis=1, keepdims=True)
    i2 = jnp.min(jnp.where(ml >= v2, idx, E), axis=1, keepdims=True)
    sel2 = idx == i2
    e2 = jnp.exp(v2 - v1)
    g1 = 1.0 / (1.0 + e2)
    g2 = e2 * g1
    gfull = jnp.where(sel1, g1, 0.0) + jnp.where(sel2, g2, 0.0)
    ei = jax.lax.broadcasted_iota(jnp.int32, (E, ER), 0)
    ej = jax.lax.broadcasted_iota(jnp.int32, (E, ER), 1)
    expand = (ei == ej // R).astype(jnp.float32)
    gexp = _dot(gfull, expand, _HI)
    ha = _dot(xt, a2d_ref[...])
    ii = jax.lax.broadcasted_iota(jnp.int32, (ER, ER), 0)
    jj = jax.lax.broadcasted_iota(jnp.int32, (ER, ER), 1)
    avg = jnp.where(ii // R == jj // R, 1.0 / R, 0.0)
    mu = _dot(ha, avg, _HI)
    dev = ha - mu
    var = _dot(dev * dev, avg, _HI)
    hn = dev * jax.lax.rsqrt(var + EPS)
    hc = (hn * gain_ref[...] + bias_ref[...]) * gexp
    out_ref[...] = _dot(hc, b2d_ref[...])


@functools.partial(jax.jit, static_argnames=())
def kernel(x, W1, b1, W2, b2, A, Bm, scaling, ln_g, ln_b):
    T = B * S
    xf = x.reshape(T, D)
    a2d = jnp.transpose(A, (1, 0, 2)).reshape(D, ER)
    b2d = Bm.reshape(ER, DO)
    gain = (ln_g * scaling[:, None]).reshape(1, ER)
    bias = (ln_b * scaling[:, None]).reshape(1, ER)
    b1r = b1.reshape(1, H)
    b2r = b2.reshape(1, E)

    grid = (T // TT,)
    out = pl.pallas_call(
        _fused_kernel,
        grid=grid,
        in_specs=[
            pl.BlockSpec((TT, D), lambda i: (i, 0)),
            pl.BlockSpec((D, H), lambda i: (0, 0)),
            pl.BlockSpec((1, H), lambda i: (0, 0)),
            pl.BlockSpec((H, E), lambda i: (0, 0)),
            pl.BlockSpec((1, E), lambda i: (0, 0)),
            pl.BlockSpec((D, ER), lambda i: (0, 0)),
            pl.BlockSpec((1, ER), lambda i: (0, 0)),
            pl.BlockSpec((1, ER), lambda i: (0, 0)),
            pl.BlockSpec((ER, DO), lambda i: (0, 0)),
        ],
        out_specs=pl.BlockSpec((TT, DO), lambda i: (i, 0)),
        out_shape=jax.ShapeDtypeStruct((T, DO), jnp.float32),
    )(xf, W1, b1r, W2, b2r, a2d, gain, bias, b2d)
    return out.reshape(B, S, DO)


# speedup vs baseline: 1.0164x; 1.0164x over previous
import functools

import jax
import jax.numpy as jnp
from jax.experimental import pallas as pl

B, S, D = 1, 2048, 2048
H = D // 2
E = 8
K = 2
R = 8
DO = 2048
EPS = 1e-5
ER = E * R
TT = 512

_HI = jax.lax.Precision.HIGHEST
_DEF = jax.lax.Precision.DEFAULT


def _dot(a, b, prec=_DEF):
    return jnp.dot(a, b, precision=prec, preferred_element_type=jnp.float32)


def _fused_kernel(x_ref, w1_ref, b1_ref, w2_ref, b2_ref, a2d_ref, gain_ref,
                  bias_ref, b2d_ref, out_ref):
    xt = x_ref[...]
    h = jax.nn.gelu(_dot(xt, w1_ref[...]) + b1_ref[...])
    logits = _dot(h, w2_ref[...]) + b2_ref[...]
    lb = jax.lax.bitcast_convert_type(logits, jnp.int32)
    mono = lb ^ jax.lax.shift_right_arithmetic(lb, 31) & 0x7FFFFFFF
    idx = jax.lax.broadcasted_iota(jnp.int32, (TT, E), 1)
    key = (mono & ~7) + (7 - idx)
    k1 = jnp.max(key, axis=1, keepdims=True)
    sel1 = key == k1
    k2 = jnp.max(jnp.where(sel1, jnp.iinfo(jnp.int32).min, key),
                 axis=1, keepdims=True)
    sel2 = key == k2

    def _decode(k):
        m = k & ~7
        return jax.lax.bitcast_convert_type(
            m ^ jax.lax.shift_right_arithmetic(m, 31) & 0x7FFFFFFF,
            jnp.float32)

    e2 = jnp.exp(_decode(k2) - _decode(k1))
    g1 = 1.0 / (1.0 + e2)
    g2 = e2 * g1
    gfull = jnp.where(sel1, g1, 0.0) + jnp.where(sel2, g2, 0.0)
    ei = jax.lax.broadcasted_iota(jnp.int32, (E, ER), 0)
    ej = jax.lax.broadcasted_iota(jnp.int32, (E, ER), 1)
    expand = (ei == ej // R).astype(jnp.float32)
    gexp = _dot(gfull, expand, _HI)
    ha = _dot(xt, a2d_ref[...])
    ii = jax.lax.broadcasted_iota(jnp.int32, (ER, ER), 0)
    jj = jax.lax.broadcasted_iota(jnp.int32, (ER, ER), 1)
    avg = jnp.where(ii // R == jj // R, 1.0 / R, 0.0)
    mu = _dot(ha, avg, _HI)
    dev = ha - mu
    var = _dot(dev * dev, avg, _HI)
    hn = dev * jax.lax.rsqrt(var + EPS)
    hc = (hn * gain_ref[...] + bias_ref[...]) * gexp
    out_ref[...] = _dot(hc, b2d_ref[...])


@functools.partial(jax.jit, static_argnames=())
def kernel(x, W1, b1, W2, b2, A, Bm, scaling, ln_g, ln_b):
    T = B * S
    xf = x.reshape(T, D)
    a2d = jnp.transpose(A, (1, 0, 2)).reshape(D, ER)
    b2d = Bm.reshape(ER, DO)
    gain = (ln_g * scaling[:, None]).reshape(1, ER)
    bias = (ln_b * scaling[:, None]).reshape(1, ER)
    b1r = b1.reshape(1, H)
    b2r = b2.reshape(1, E)

    grid = (T // TT,)
    out = pl.pallas_call(
        _fused_kernel,
        grid=grid,
        in_specs=[
            pl.BlockSpec((TT, D), lambda i: (i, 0)),
            pl.BlockSpec((D, H), lambda i: (0, 0)),
            pl.BlockSpec((1, H), lambda i: (0, 0)),
            pl.BlockSpec((H, E), lambda i: (0, 0)),
            pl.BlockSpec((1, E), lambda i: (0, 0)),
            pl.BlockSpec((D, ER), lambda i: (0, 0)),
            pl.BlockSpec((1, ER), lambda i: (0, 0)),
            pl.BlockSpec((1, ER), lambda i: (0, 0)),
            pl.BlockSpec((ER, DO), lambda i: (0, 0)),
        ],
        out_specs=pl.BlockSpec((TT, DO), lambda i: (i, 0)),
        out_shape=jax.ShapeDtypeStruct((T, DO), jnp.float32),
    )(xf, W1, b1r, W2, b2r, a2d, gain, bias, b2d)
    return out.reshape(B, S, DO)


# speedup vs baseline: 1.0974x; 1.0796x over previous
import functools

import jax
import jax.numpy as jnp
from jax.experimental import pallas as pl

B, S, D = 1, 2048, 2048
H = D // 2
E = 8
K = 2
R = 8
DO = 2048
EPS = 1e-5
ER = E * R
TT = 512

_HI = jax.lax.Precision.HIGHEST
_DEF = jax.lax.Precision.DEFAULT


def _dot(a, b, prec=_DEF):
    return jnp.dot(a, b, precision=prec, preferred_element_type=jnp.float32)


def _fused_kernel(x_ref, w1_ref, b1_ref, w2_ref, b2_ref, a2d_ref, gain_ref,
                  bias_ref, b2d_ref, out_ref):
    xt = x_ref[...]
    h = jax.nn.gelu(_dot(xt, w1_ref[...]) + b1_ref[...])
    logits = _dot(h, w2_ref[...]) + b2_ref[...]
    lb = jax.lax.bitcast_convert_type(logits, jnp.int32)
    mono = lb ^ jax.lax.shift_right_arithmetic(lb, 31) & 0x7FFFFFFF
    idx = jax.lax.broadcasted_iota(jnp.int32, (TT, E), 1)
    key = (mono & ~7) + (7 - idx)
    k1 = jnp.max(key, axis=1, keepdims=True)
    sel1 = key == k1
    k2 = jnp.max(jnp.where(sel1, jnp.iinfo(jnp.int32).min, key),
                 axis=1, keepdims=True)
    sel2 = key == k2

    def _decode(k):
        m = k & ~7
        return jax.lax.bitcast_convert_type(
            m ^ jax.lax.shift_right_arithmetic(m, 31) & 0x7FFFFFFF,
            jnp.float32)

    e2 = jnp.exp(_decode(k2) - _decode(k1))
    g1 = 1.0 / (1.0 + e2)
    g2 = e2 * g1
    gfull = jnp.where(sel1, g1, 0.0) + jnp.where(sel2, g2, 0.0)
    ei = jax.lax.broadcasted_iota(jnp.int32, (E, ER), 0)
    ej = jax.lax.broadcasted_iota(jnp.int32, (E, ER), 1)
    expand = (ei == ej // R).astype(jnp.float32)
    gexp = _dot(gfull, expand)
    ha = _dot(xt, a2d_ref[...])
    ii = jax.lax.broadcasted_iota(jnp.int32, (ER, ER), 0)
    jj = jax.lax.broadcasted_iota(jnp.int32, (ER, ER), 1)
    avg = jnp.where(ii // R == jj // R, 1.0 / R, 0.0)
    mu = _dot(ha, avg, _HI)
    dev = ha - mu
    var = _dot(dev * dev, avg)
    hn = dev * jax.lax.rsqrt(var + EPS)
    hc = (hn * gain_ref[...] + bias_ref[...]) * gexp
    out_ref[...] = _dot(hc, b2d_ref[...])


@functools.partial(jax.jit, static_argnames=())
def kernel(x, W1, b1, W2, b2, A, Bm, scaling, ln_g, ln_b):
    T = B * S
    xf = x.reshape(T, D)
    a2d = jnp.transpose(A, (1, 0, 2)).reshape(D, ER)
    b2d = Bm.reshape(ER, DO)
    gain = (ln_g * scaling[:, None]).reshape(1, ER)
    bias = (ln_b * scaling[:, None]).reshape(1, ER)
    b1r = b1.reshape(1, H)
    b2r = b2.reshape(1, E)

    grid = (T // TT,)
    out = pl.pallas_call(
        _fused_kernel,
        grid=grid,
        in_specs=[
            pl.BlockSpec((TT, D), lambda i: (i, 0)),
            pl.BlockSpec((D, H), lambda i: (0, 0)),
            pl.BlockSpec((1, H), lambda i: (0, 0)),
            pl.BlockSpec((H, E), lambda i: (0, 0)),
            pl.BlockSpec((1, E), lambda i: (0, 0)),
            pl.BlockSpec((D, ER), lambda i: (0, 0)),
            pl.BlockSpec((1, ER), lambda i: (0, 0)),
            pl.BlockSpec((1, ER), lambda i: (0, 0)),
            pl.BlockSpec((ER, DO), lambda i: (0, 0)),
        ],
        out_specs=pl.BlockSpec((TT, DO), lambda i: (i, 0)),
        out_shape=jax.ShapeDtypeStruct((T, DO), jnp.float32),
    )(xf, W1, b1r, W2, b2r, a2d, gain, bias, b2d)
    return out.reshape(B, S, DO)


# speedup vs baseline: 1.1199x; 1.0205x over previous
import functools

import jax
import jax.numpy as jnp
from jax.experimental import pallas as pl

B, S, D = 1, 2048, 2048
H = D // 2
E = 8
K = 2
R = 8
DO = 2048
EPS = 1e-5
ER = E * R
TT = 512

_HI = jax.lax.Precision.HIGHEST
_DEF = jax.lax.Precision.DEFAULT


def _dot(a, b, prec=_DEF):
    return jnp.dot(a, b, precision=prec, preferred_element_type=jnp.float32)


def _fused_kernel(x_ref, w1_ref, b1_ref, w2_ref, b2_ref, a2d_ref, gain_ref,
                  bias_ref, b2d_ref, out_ref):
    xt = x_ref[...]
    h = jax.nn.gelu(_dot(xt, w1_ref[...]) + b1_ref[...])
    logits = _dot(h, w2_ref[...]) + b2_ref[...]
    lb = jax.lax.bitcast_convert_type(logits, jnp.int32)
    mono = lb ^ jax.lax.shift_right_arithmetic(lb, 31) & 0x7FFFFFFF
    idx = jax.lax.broadcasted_iota(jnp.int32, (TT, E), 1)
    key = (mono & ~7) + (7 - idx)
    k1 = jnp.max(key, axis=1, keepdims=True)
    sel1 = key == k1
    k2 = jnp.max(jnp.where(sel1, jnp.iinfo(jnp.int32).min, key),
                 axis=1, keepdims=True)
    sel2 = key == k2

    def _decode(k):
        m = k & ~7
        return jax.lax.bitcast_convert_type(
            m ^ jax.lax.shift_right_arithmetic(m, 31) & 0x7FFFFFFF,
            jnp.float32)

    e2 = jnp.exp(_decode(k2) - _decode(k1))
    g1 = 1.0 / (1.0 + e2)
    g2 = e2 * g1
    gfull = jnp.where(sel1, g1, 0.0) + jnp.where(sel2, g2, 0.0)
    ei = jax.lax.broadcasted_iota(jnp.int32, (E, ER), 0)
    ej = jax.lax.broadcasted_iota(jnp.int32, (E, ER), 1)
    expand = (ei == ej // R).astype(jnp.float32)
    gexp = _dot(gfull, expand)
    ha = _dot(xt, a2d_ref[...])
    ii = jax.lax.broadcasted_iota(jnp.int32, (ER, ER), 0)
    jj = jax.lax.broadcasted_iota(jnp.int32, (ER, ER), 1)
    avg = jnp.where(ii // R == jj // R, 1.0 / R, 0.0)
    mu = _dot(ha, avg)
    dev = ha - mu
    var = _dot(dev * dev, avg)
    hn = dev * jax.lax.rsqrt(var + EPS)
    hc = (hn * gain_ref[...] + bias_ref[...]) * gexp
    out_ref[...] = _dot(hc, b2d_ref[...])


@functools.partial(jax.jit, static_argnames=())
def kernel(x, W1, b1, W2, b2, A, Bm, scaling, ln_g, ln_b):
    T = B * S
    xf = x.reshape(T, D)
    a2d = jnp.transpose(A, (1, 0, 2)).reshape(D, ER)
    b2d = Bm.reshape(ER, DO)
    gain = (ln_g * scaling[:, None]).reshape(1, ER)
    bias = (ln_b * scaling[:, None]).reshape(1, ER)
    b1r = b1.reshape(1, H)
    b2r = b2.reshape(1, E)

    grid = (T // TT,)
    out = pl.pallas_call(
        _fused_kernel,
        grid=grid,
        in_specs=[
            pl.BlockSpec((TT, D), lambda i: (i, 0)),
            pl.BlockSpec((D, H), lambda i: (0, 0)),
            pl.BlockSpec((1, H), lambda i: (0, 0)),
            pl.BlockSpec((H, E), lambda i: (0, 0)),
            pl.BlockSpec((1, E), lambda i: (0, 0)),
            pl.BlockSpec((D, ER), lambda i: (0, 0)),
            pl.BlockSpec((1, ER), lambda i: (0, 0)),
            pl.BlockSpec((1, ER), lambda i: (0, 0)),
            pl.BlockSpec((ER, DO), lambda i: (0, 0)),
        ],
        out_specs=pl.BlockSpec((TT, DO), lambda i: (i, 0)),
        out_shape=jax.ShapeDtypeStruct((T, DO), jnp.float32),
    )(xf, W1, b1r, W2, b2r, a2d, gain, bias, b2d)
    return out.reshape(B, S, DO)
